# baseline (speedup 1.0000x reference)
import jax
import jax.numpy as jnp
from jax.experimental import pallas as pl
from jax.experimental.pallas import tpu as pltpu

ALPHA = -50.0


def _body(x_ref, bins_ref, o_ref):
    v = x_ref[...]
    b = [bins_ref[k] for k in range(4)]
    c1 = v < b[1]
    c2 = v < b[2]
    lo = jnp.where(c1, b[0], jnp.where(c2, b[1], b[2]))
    hi = jnp.where(c1, b[1], jnp.where(c2, b[2], b[3]))
    z = ALPHA * (lo + hi - (v + v))
    w = 1.0 / (1.0 + jnp.exp(-z))
    o_ref[...] = lo + (hi - lo) * w


def kernel(x, bins):
    B, F, C = x.shape
    xt = jnp.transpose(x, (1, 0, 2))
    blk = 256
    grid = (B // blk,)
    out = pl.pallas_call(
        _body,
        grid=grid,
        in_specs=[
            pl.BlockSpec((F, blk, C), lambda i: (0, i, 0)),
            pl.BlockSpec(memory_space=pltpu.SMEM),
        ],
        out_specs=pl.BlockSpec((F, blk, C), lambda i: (0, i, 0)),
        out_shape=jax.ShapeDtypeStruct((F, B, C), x.dtype),
    )(xt, bins)
    return jnp.transpose(out, (1, 0, 2))
